# baseline (speedup 1.0000x reference)
import functools

import jax
import jax.numpy as jnp
from jax.experimental import pallas as pl
from jax.experimental.pallas import tpu as pltpu

NODE_DIM = 128
EDGE_DIM = 16
HID = 272


def _lrelu(x, s):
    return jnp.where(x > 0, x, s * x)


def _pick_tile(n, cands):
    for c in cands:
        if n % c == 0:
            return c
    return n


def _edge_stage(h_src, h_dst, edge_feats, p):
    E = h_src.shape[0]
    T = _pick_tile(E, (2560, 1280, 640, 512, 320, 160, 80, 40, 8))
    grid = (E // T,)
    w1 = p['mb_w1']
    w1s = w1[:NODE_DIM]
    w1d = w1[NODE_DIM:2 * NODE_DIM]
    w1e = w1[2 * NODE_DIM:2 * NODE_DIM + EDGE_DIM]
    wih_t = p['egru_wih'].T
    whh_t = p['egru_whh'].T
    bih = p['egru_bih'].reshape(1, -1)
    bhh = p['egru_bhh'].reshape(1, -1)

    def body(hs_ref, hd_ref, ef_ref, w1s_ref, w1d_ref, w1e_ref, b1_ref,
             g_ref, bb_ref, w2_ref, b2_ref, wih_ref, whh_ref, bih_ref,
             bhh_ref, resw_ref, resb_ref, enew_ref):
        hs = hs_ref[...]
        hd = hd_ref[...]
        ef = ef_ref[...]
        h = (hs @ w1s_ref[...] + hd @ w1d_ref[...] + ef @ w1e_ref[...]
             + b1_ref[...])
        mu = jnp.mean(h, axis=-1, keepdims=True)
        var = jnp.mean((h - mu) ** 2, axis=-1, keepdims=True)
        h = (h - mu) * jax.lax.rsqrt(var + 1e-5) * g_ref[...] + bb_ref[...]
        h = _lrelu(h, 0.2)
        x = h @ w2_ref[...] + b2_ref[...]
        gi = x @ wih_ref[...] + bih_ref[...]
        gh = ef @ whh_ref[...] + bhh_ref[...]
        i_r, i_z, i_n = gi[:, :16], gi[:, 16:32], gi[:, 32:48]
        h_r, h_z, h_n = gh[:, :16], gh[:, 16:32], gh[:, 32:48]
        r = jax.nn.sigmoid(i_r + h_r)
        z = jax.nn.sigmoid(i_z + h_z)
        n = jnp.tanh(i_n + r * h_n)
        e0 = (1.0 - z) * n + z * ef
        e_new = _lrelu(e0 + ef @ resw_ref[...] + resb_ref[...], 0.01)
        enew_ref[...] = e_new

    def rowspec(d):
        return pl.BlockSpec((T, d), lambda i: (i, 0))

    def wspec(a):
        return pl.BlockSpec(a.shape, lambda i: (0,) * a.ndim)

    ws = [w1s, w1d, w1e, p['mb_b1'].reshape(1, -1),
          p['mb_ln_g'].reshape(1, -1), p['mb_ln_b'].reshape(1, -1),
          p['mb_w2'], p['mb_b2'].reshape(1, -1), wih_t, whh_t, bih, bhh,
          p['eres_w'], p['eres_b'].reshape(1, -1)]
    return pl.pallas_call(
        body,
        grid=grid,
        in_specs=[rowspec(NODE_DIM), rowspec(NODE_DIM), rowspec(EDGE_DIM)]
                 + [wspec(a) for a in ws],
        out_specs=rowspec(EDGE_DIM),
        out_shape=jax.ShapeDtypeStruct((E, EDGE_DIM), jnp.float32),
    )(h_src, h_dst, edge_feats, *ws)


def _node_stage(s_t, mx_t, node_feats, p):
    N = node_feats.shape[0]
    T = _pick_tile(N, (2048, 1024, 512, 256, 128))
    grid = (N // T,)
    D = NODE_DIM + EDGE_DIM
    w1 = p['mb_w1']
    w1s = w1[:D]
    w1m = w1[D:2 * D]
    wih_t = p['agru_wih'].T
    whh_t = p['agru_whh'].T
    bih = p['agru_bih'].reshape(1, -1)
    bhh = p['agru_bhh'].reshape(1, -1)

    def body(st_ref, mt_ref, nf_ref, w1s_ref, w1m_ref, b1_ref, g_ref, bb_ref,
             w2_ref, b2_ref, wih_ref, whh_ref, bih_ref, bhh_ref, resw_ref,
             resb_ref, hnew_ref):
        st = st_ref[...]
        mt = mt_ref[...]
        mt = jnp.where(jnp.isfinite(mt), mt, 0.0)
        nf = nf_ref[...]
        dn = (((0,), (0,)), ((), ()))
        h = (jax.lax.dot_general(st, w1s_ref[...], dn,
                                 preferred_element_type=jnp.float32)
             + jax.lax.dot_general(mt, w1m_ref[...], dn,
                                   preferred_element_type=jnp.float32)
             + b1_ref[...])
        mu = jnp.mean(h, axis=-1, keepdims=True)
        var = jnp.mean((h - mu) ** 2, axis=-1, keepdims=True)
        h = (h - mu) * jax.lax.rsqrt(var + 1e-5) * g_ref[...] + bb_ref[...]
        h = _lrelu(h, 0.2)
        x = h @ w2_ref[...] + b2_ref[...]
        gi = x @ wih_ref[...] + bih_ref[...]
        gh = nf @ whh_ref[...] + bhh_ref[...]
        K = NODE_DIM
        i_r, i_z, i_n = gi[:, :K], gi[:, K:2 * K], gi[:, 2 * K:3 * K]
        h_r, h_z, h_n = gh[:, :K], gh[:, K:2 * K], gh[:, 2 * K:3 * K]
        r = jax.nn.sigmoid(i_r + h_r)
        z = jax.nn.sigmoid(i_z + h_z)
        n = jnp.tanh(i_n + r * h_n)
        h0 = (1.0 - z) * n + z * nf
        h_new = _lrelu(h0 + nf @ resw_ref[...] + resb_ref[...], 0.01)
        hnew_ref[...] = h_new

    def wspec(a):
        return pl.BlockSpec(a.shape, lambda i: (0,) * a.ndim)

    ws = [w1s, w1m, p['mb_b1'].reshape(1, -1), p['mb_ln_g'].reshape(1, -1),
          p['mb_ln_b'].reshape(1, -1), p['mb_w2'],
          p['mb_b2'].reshape(1, -1), wih_t, whh_t, bih, bhh,
          p['ares_w'], p['ares_b'].reshape(1, -1)]
    return pl.pallas_call(
        body,
        grid=grid,
        in_specs=[pl.BlockSpec((D, T), lambda i: (0, i)),
                  pl.BlockSpec((D, T), lambda i: (0, i)),
                  pl.BlockSpec((T, NODE_DIM), lambda i: (i, 0))]
                 + [wspec(a) for a in ws],
        out_specs=pl.BlockSpec((T, NODE_DIM), lambda i: (i, 0)),
        out_shape=jax.ShapeDtypeStruct((N, NODE_DIM), jnp.float32),
    )(s_t, mx_t, node_feats, *ws)


def kernel(node_feats, edge_feats, params, edge_index):
    src = edge_index[0]
    dst = edge_index[1]
    h_src = node_feats[src]
    h_dst = node_feats[dst]
    e_new = _edge_stage(h_src, h_dst, edge_feats, params)
    n = node_feats.shape[0]
    m = jnp.concatenate([h_src, e_new], axis=1)
    s = jax.ops.segment_sum(m, dst, num_segments=n)
    mx = jax.ops.segment_max(m, dst, num_segments=n)
    npad = ((n + 2047) // 2048) * 2048
    s_t = jnp.pad(s.T, ((0, 0), (0, npad - n)))
    mx_t = jnp.pad(mx.T, ((0, 0), (0, npad - n)))
    nf_pad = jnp.pad(node_feats, ((0, npad - n), (0, 0)))
    h_new = _node_stage(s_t, mx_t, nf_pad, params)[:n]
    return (h_new, e_new)
